# baseline (speedup 1.0000x reference)
import jax
import jax.numpy as jnp
from jax.experimental import pallas as pl
from jax.experimental.pallas import tpu as pltpu
from jax.experimental.pallas import tpu_sc as plsc

_D = 256
_K = 8192
_TILE = 256
_TEMP_INV_LOG2E = 20.0 * 1.4426950408889634
_GATHER_WINDOW = 128


def _vq_body(z_ref, w_ref, idx_ref, probs_ref, dmin_ref, w2_ref, iota_ref):
    @pl.when(pl.program_id(0) == 0)
    def _():
        w = w_ref[...]
        w2_ref[0, :] = jnp.sum(w * w, axis=1)
        ii = jax.lax.broadcasted_iota(jnp.int32, (_TILE, _K), 1)
        iota_ref[...] = ii.astype(jnp.float32)

    z = z_ref[...]
    z2 = jnp.sum(z * z, axis=1, keepdims=True)
    mm = jax.lax.dot_general(z, w_ref[...], (((1,), (1,)), ((), ())),
                             preferred_element_type=jnp.float32)
    d = (z2 + w2_ref[0, :][None, :]) - 2.0 * mm
    dmin = jnp.min(d, axis=1, keepdims=True)
    idxf = jnp.min(jnp.where(d == dmin, iota_ref[...], float(_K)), axis=1)
    idx = idxf.astype(jnp.int32)
    p = jnp.exp2((dmin - d) * _TEMP_INV_LOG2E)
    l_inv = 1.0 / jnp.sum(p, axis=1, keepdims=True)
    pm = (p * l_inv).reshape(_TILE // 8, 8, _K)
    probs_ref[0, :, :] = jnp.sum(pm, axis=0)
    idx_ref[0, 0, :] = idx
    dmin_ref[0, 0, :] = dmin[:, 0]


def _vq_distances(flat_z, weight):
    n = flat_z.shape[0]
    grid = n // _TILE
    return pl.pallas_call(
        _vq_body,
        grid=(grid,),
        in_specs=[
            pl.BlockSpec((_TILE, _D), lambda i: (i, 0)),
            pl.BlockSpec((_K, _D), lambda i: (0, 0)),
        ],
        out_specs=[
            pl.BlockSpec((1, 1, _TILE), lambda i: (i, 0, 0)),
            pl.BlockSpec((1, 8, _K), lambda i: (i, 0, 0)),
            pl.BlockSpec((1, 1, _TILE), lambda i: (i, 0, 0)),
        ],
        out_shape=[
            jax.ShapeDtypeStruct((grid, 1, _TILE), jnp.int32),
            jax.ShapeDtypeStruct((grid, 8, _K), jnp.float32),
            jax.ShapeDtypeStruct((grid, 1, _TILE), jnp.float32),
        ],
        scratch_shapes=[pltpu.VMEM((1, _K), jnp.float32),
                        pltpu.VMEM((_TILE, _K), jnp.float32)],
        compiler_params=pltpu.CompilerParams(
            dimension_semantics=("arbitrary",),
        ),
    )(flat_z, weight)


def _sc_gather(weight, idx):
    n = idx.shape[0]
    ind = idx.reshape(1, n)
    mesh = plsc.VectorSubcoreMesh(core_axis_name="core",
                                  subcore_axis_name="subcore")

    @pl.kernel(out_type=jax.ShapeDtypeStruct((n, _D), weight.dtype),
               mesh=mesh)
    def kern(w_hbm, i_hbm, o_hbm):
        def body(i_vmem, o_vmem):
            pltpu.sync_copy(w_hbm.at[i_vmem.at[0]], o_vmem)

        pltpu.emit_pipeline(
            body,
            grid=(n // _GATHER_WINDOW,),
            in_specs=[pl.BlockSpec((1, _GATHER_WINDOW),
                                   index_map=lambda i: (0, i))],
            out_specs=[pl.BlockSpec((_GATHER_WINDOW, _D),
                                    index_map=lambda i: (i, 0))],
            core_axis_name=("core", "subcore"),
            dimension_semantics=(pltpu.PARALLEL,),
        )(i_hbm, o_hbm)

    return kern(weight, ind)


def kernel(z, weight):
    B, T, D = z.shape
    n = B * T
    flat_z = z.reshape(n, D)
    idx3, probs3, dmin3 = _vq_distances(flat_z, weight)
    idx = idx3.reshape(n)
    avg_soft_probs = jnp.sum(probs3, axis=(0, 1)) / n
    vq_loss = 1.25 * (jnp.sum(dmin3) / (n * D))
    quantized = _sc_gather(weight, idx).reshape(B, T, D)
    quantized_st = z + (quantized - z)
    encoding_indices = idx.reshape(B, T)
    return (quantized_st, vq_loss, encoding_indices, avg_soft_probs)


# speedup vs baseline: 1.0555x; 1.0555x over previous
import jax
import jax.numpy as jnp
from jax.experimental import pallas as pl
from jax.experimental.pallas import tpu as pltpu
from jax.experimental.pallas import tpu_sc as plsc

_D = 256
_K = 8192
_TILE = 256
_TEMP_INV_LOG2E = 20.0 * 1.4426950408889634
_GATHER_WINDOW = 128


def _prep_body(w_ref, w2_ref, iota_ref):
    w = w_ref[...]
    w2_ref[0, :] = jnp.sum(w * w, axis=1)
    ii = jax.lax.broadcasted_iota(jnp.int32, (1, _K), 1)
    iota_ref[...] = ii.astype(jnp.float32)


def _vq_prep(weight):
    return pl.pallas_call(
        _prep_body,
        out_shape=[
            jax.ShapeDtypeStruct((1, _K), jnp.float32),
            jax.ShapeDtypeStruct((1, _K), jnp.float32),
        ],
    )(weight)


def _vq_body(z_ref, w_ref, w2_ref, iota_ref, idx_ref, probs_ref, dmin_ref):
    z = z_ref[...]
    z2 = jnp.sum(z * z, axis=1, keepdims=True)
    mm = jax.lax.dot_general(z, w_ref[...], (((1,), (1,)), ((), ())),
                             preferred_element_type=jnp.float32)
    d = (z2 + w2_ref[0, :][None, :]) - 2.0 * mm
    dmin = jnp.min(d, axis=1, keepdims=True)
    idxf = jnp.min(jnp.where(d == dmin, iota_ref[0, :][None, :], float(_K)),
                   axis=1)
    idx = idxf.astype(jnp.int32)
    p = jnp.exp2((dmin - d) * _TEMP_INV_LOG2E)
    l_inv = 1.0 / jnp.sum(p, axis=1, keepdims=True)
    probs_ref[0, 0, :] = jax.lax.dot_general(
        l_inv, p, (((0,), (0,)), ((), ())),
        preferred_element_type=jnp.float32)[0, :]
    idx_ref[0, 0, :] = idx
    dmin_ref[0, 0, :] = dmin[:, 0]


def _vq_distances(flat_z, weight):
    n = flat_z.shape[0]
    grid = n // _TILE
    w2, iota = _vq_prep(weight)
    return pl.pallas_call(
        _vq_body,
        grid=(grid,),
        in_specs=[
            pl.BlockSpec((_TILE, _D), lambda i: (i, 0)),
            pl.BlockSpec((_K, _D), lambda i: (0, 0)),
            pl.BlockSpec((1, _K), lambda i: (0, 0)),
            pl.BlockSpec((1, _K), lambda i: (0, 0)),
        ],
        out_specs=[
            pl.BlockSpec((1, 1, _TILE), lambda i: (i, 0, 0)),
            pl.BlockSpec((1, 1, _K), lambda i: (i, 0, 0)),
            pl.BlockSpec((1, 1, _TILE), lambda i: (i, 0, 0)),
        ],
        out_shape=[
            jax.ShapeDtypeStruct((grid, 1, _TILE), jnp.int32),
            jax.ShapeDtypeStruct((grid, 1, _K), jnp.float32),
            jax.ShapeDtypeStruct((grid, 1, _TILE), jnp.float32),
        ],
        compiler_params=pltpu.CompilerParams(
            dimension_semantics=("arbitrary",),
        ),
    )(flat_z, weight, w2, iota)


def _sc_gather(weight, idx):
    n = idx.shape[0]
    ind = idx.reshape(1, n)
    mesh = plsc.VectorSubcoreMesh(core_axis_name="core",
                                  subcore_axis_name="subcore")

    @pl.kernel(out_type=jax.ShapeDtypeStruct((n, _D), weight.dtype),
               mesh=mesh)
    def kern(w_hbm, i_hbm, o_hbm):
        def body(i_vmem, o_vmem):
            pltpu.sync_copy(w_hbm.at[i_vmem.at[0]], o_vmem)

        pltpu.emit_pipeline(
            body,
            grid=(n // _GATHER_WINDOW,),
            in_specs=[pl.BlockSpec((1, _GATHER_WINDOW),
                                   index_map=lambda i: (0, i))],
            out_specs=[pl.BlockSpec((_GATHER_WINDOW, _D),
                                    index_map=lambda i: (i, 0))],
            core_axis_name=("core", "subcore"),
            dimension_semantics=(pltpu.PARALLEL,),
        )(i_hbm, o_hbm)

    return kern(weight, ind)


def kernel(z, weight):
    B, T, D = z.shape
    n = B * T
    flat_z = z.reshape(n, D)
    idx3, probs3, dmin3 = _vq_distances(flat_z, weight)
    idx = idx3.reshape(n)
    avg_soft_probs = jnp.sum(probs3.reshape(-1, _K), axis=0) / n
    vq_loss = 1.25 * (jnp.sum(dmin3) / (n * D))
    quantized = _sc_gather(weight, idx).reshape(B, T, D)
    quantized_st = z + (quantized - z)
    encoding_indices = idx.reshape(B, T)
    return (quantized_st, vq_loss, encoding_indices, avg_soft_probs)


# speedup vs baseline: 1.1065x; 1.0483x over previous
import jax
import jax.numpy as jnp
from jax.experimental import pallas as pl
from jax.experimental.pallas import tpu as pltpu
from jax.experimental.pallas import tpu_sc as plsc

_D = 256
_K = 8192
_TILE = 512
_TEMP_INV_LOG2E = 20.0 * 1.4426950408889634
_GATHER_WINDOW = 128


def _prep_body(w_ref, w2_ref, iota_ref):
    w = w_ref[...]
    w2_ref[0, :] = jnp.sum(w * w, axis=1)
    ii = jax.lax.broadcasted_iota(jnp.int32, (1, _K), 1)
    iota_ref[...] = ii.astype(jnp.float32)


def _vq_prep(weight):
    return pl.pallas_call(
        _prep_body,
        out_shape=[
            jax.ShapeDtypeStruct((1, _K), jnp.float32),
            jax.ShapeDtypeStruct((1, _K), jnp.float32),
        ],
    )(weight)


def _vq_body(z_ref, w_ref, w2_ref, iota_ref, idx_ref, probs_ref, dmin_ref):
    z = z_ref[...]
    z2 = jnp.sum(z * z, axis=1, keepdims=True)
    mm = jax.lax.dot_general(z, w_ref[...], (((1,), (1,)), ((), ())),
                             preferred_element_type=jnp.float32)
    d = (z2 + w2_ref[0, :][None, :]) - 2.0 * mm
    dmin = jnp.min(d, axis=1, keepdims=True)
    idxf = jnp.min(jnp.where(d == dmin, iota_ref[0, :][None, :], float(_K)),
                   axis=1)
    idx = idxf.astype(jnp.int32)
    p = jnp.exp2((dmin - d) * _TEMP_INV_LOG2E)
    l_inv = 1.0 / jnp.sum(p, axis=1, keepdims=True)
    probs_ref[0, 0, :] = jax.lax.dot_general(
        l_inv, p, (((0,), (0,)), ((), ())),
        preferred_element_type=jnp.float32)[0, :]
    idx_ref[0, 0, :] = idx
    dmin_ref[0, 0, :] = dmin[:, 0]


def _vq_distances(flat_z, weight):
    n = flat_z.shape[0]
    grid = n // _TILE
    w2, iota = _vq_prep(weight)
    return pl.pallas_call(
        _vq_body,
        grid=(grid,),
        in_specs=[
            pl.BlockSpec((_TILE, _D), lambda i: (i, 0)),
            pl.BlockSpec((_K, _D), lambda i: (0, 0)),
            pl.BlockSpec((1, _K), lambda i: (0, 0)),
            pl.BlockSpec((1, _K), lambda i: (0, 0)),
        ],
        out_specs=[
            pl.BlockSpec((1, 1, _TILE), lambda i: (i, 0, 0)),
            pl.BlockSpec((1, 1, _K), lambda i: (i, 0, 0)),
            pl.BlockSpec((1, 1, _TILE), lambda i: (i, 0, 0)),
        ],
        out_shape=[
            jax.ShapeDtypeStruct((grid, 1, _TILE), jnp.int32),
            jax.ShapeDtypeStruct((grid, 1, _K), jnp.float32),
            jax.ShapeDtypeStruct((grid, 1, _TILE), jnp.float32),
        ],
        compiler_params=pltpu.CompilerParams(
            dimension_semantics=("arbitrary",),
        ),
    )(flat_z, weight, w2, iota)


def _sc_gather(weight, idx):
    n = idx.shape[0]
    ind = idx.reshape(1, n)
    mesh = plsc.VectorSubcoreMesh(core_axis_name="core",
                                  subcore_axis_name="subcore")

    @pl.kernel(out_type=jax.ShapeDtypeStruct((n, _D), weight.dtype),
               mesh=mesh)
    def kern(w_hbm, i_hbm, o_hbm):
        def body(i_vmem, o_vmem):
            pltpu.sync_copy(w_hbm.at[i_vmem.at[0]], o_vmem)

        pltpu.emit_pipeline(
            body,
            grid=(n // _GATHER_WINDOW,),
            in_specs=[pl.BlockSpec((1, _GATHER_WINDOW),
                                   index_map=lambda i: (0, i))],
            out_specs=[pl.BlockSpec((_GATHER_WINDOW, _D),
                                    index_map=lambda i: (i, 0))],
            core_axis_name=("core", "subcore"),
            dimension_semantics=(pltpu.PARALLEL,),
        )(i_hbm, o_hbm)

    return kern(weight, ind)


def kernel(z, weight):
    B, T, D = z.shape
    n = B * T
    flat_z = z.reshape(n, D)
    idx3, probs3, dmin3 = _vq_distances(flat_z, weight)
    idx = idx3.reshape(n)
    avg_soft_probs = jnp.sum(probs3.reshape(-1, _K), axis=0) / n
    vq_loss = 1.25 * (jnp.sum(dmin3) / (n * D))
    quantized = _sc_gather(weight, idx).reshape(B, T, D)
    quantized_st = z + (quantized - z)
    encoding_indices = idx.reshape(B, T)
    return (quantized_st, vq_loss, encoding_indices, avg_soft_probs)
